# baseline (speedup 1.0000x reference)
import jax
import jax.numpy as jnp
from jax import lax
from jax.experimental import pallas as pl
from jax.experimental.pallas import tpu as pltpu
from jax.experimental.pallas import tpu_sc as plsc

_N_FIELDS = 26
_VOCAB = 100000
_EMBED = 32
_BATCH = 16384

_NC = 2
_NS = 16
_L = 16
_HALF = _BATCH // 2


def _embed_body(xt_hbm, tbl_hbm, out_hbm, row_v, idx_v, out_v, sem_r, sem_x):
    e = lax.axis_index("s") * _NC + lax.axis_index("c")

    for i in range(_N_FIELDS):
        row_cp = pltpu.async_copy(tbl_hbm.at[i * _EMBED + e], row_v, sem_r)
        for h in range(2):
            pltpu.async_copy(
                xt_hbm.at[i, pl.ds(h * _HALF, _HALF)], idx_v, sem_x).wait()
            if h == 0:
                row_cp.wait()

            if i == 0:
                @plsc.parallel_loop(0, _HALF, _L, unroll=16)
                def _first(o):
                    g = plsc.load_gather(row_v, [idx_v[pl.ds(o, _L)]])
                    out_v[pl.ds(h * _HALF + o, _L)] = g
            else:
                @plsc.parallel_loop(0, _HALF, _L, unroll=16)
                def _accum(o):
                    g = plsc.load_gather(row_v, [idx_v[pl.ds(o, _L)]])
                    plsc.addupdate(out_v.at[pl.ds(h * _HALF + o, _L)], g)

    pltpu.sync_copy(out_v, out_hbm.at[e])


def kernel(x, tables):
    xt = x.astype(jnp.int32).T
    tbl = tables.transpose(0, 2, 1).reshape(_N_FIELDS * _EMBED, _VOCAB)

    run = pl.kernel(
        _embed_body,
        out_type=jax.ShapeDtypeStruct((_EMBED, _BATCH), jnp.float32),
        mesh=plsc.VectorSubcoreMesh(core_axis_name="c", subcore_axis_name="s",
                                    num_cores=_NC, num_subcores=_NS),
        scratch_types=[
            pltpu.VMEM((_VOCAB,), jnp.float32),
            pltpu.VMEM((_HALF,), jnp.int32),
            pltpu.VMEM((_BATCH,), jnp.float32),
            pltpu.SemaphoreType.DMA,
            pltpu.SemaphoreType.DMA,
        ],
        compiler_params=pltpu.CompilerParams(needs_layout_passes=False),
    )
    return run(xt, tbl).T


# speedup vs baseline: 1.0206x; 1.0206x over previous
import jax
import jax.numpy as jnp
from jax import lax
from jax.experimental import pallas as pl
from jax.experimental.pallas import tpu as pltpu
from jax.experimental.pallas import tpu_sc as plsc

_N_FIELDS = 26
_VOCAB = 100000
_EMBED = 32
_BATCH = 16384

_NC = 2
_NS = 16
_L = 16
_HALF = _BATCH // 2


def _embed_body(xt_hbm, tbl_hbm, out_hbm, row_v, idx_v, out_v, sem_r, sem_x):
    e = lax.axis_index("s") * _NC + lax.axis_index("c")

    for i in range(_N_FIELDS):
        row_cp = pltpu.async_copy(tbl_hbm.at[i * _EMBED + e], row_v, sem_r)
        for h in range(2):
            pltpu.async_copy(
                xt_hbm.at[i, pl.ds(h * _HALF, _HALF)], idx_v, sem_x).wait()
            if h == 0:
                row_cp.wait()

            if i == 0:
                @plsc.parallel_loop(0, _HALF, _L, unroll=8)
                def _first(o):
                    g = plsc.load_gather(row_v, [idx_v[pl.ds(o, _L)]])
                    out_v[pl.ds(h * _HALF + o, _L)] = g
            else:
                @plsc.parallel_loop(0, _HALF, _L, unroll=8)
                def _accum(o):
                    g = plsc.load_gather(row_v, [idx_v[pl.ds(o, _L)]])
                    plsc.addupdate(out_v.at[pl.ds(h * _HALF + o, _L)], g)

    pltpu.sync_copy(out_v, out_hbm.at[e])


def kernel(x, tables):
    xt = x.astype(jnp.int32).T
    tbl = tables.transpose(0, 2, 1).reshape(_N_FIELDS * _EMBED, _VOCAB)

    run = pl.kernel(
        _embed_body,
        out_type=jax.ShapeDtypeStruct((_EMBED, _BATCH), jnp.float32),
        mesh=plsc.VectorSubcoreMesh(core_axis_name="c", subcore_axis_name="s",
                                    num_cores=_NC, num_subcores=_NS),
        scratch_types=[
            pltpu.VMEM((_VOCAB,), jnp.float32),
            pltpu.VMEM((_HALF,), jnp.int32),
            pltpu.VMEM((_BATCH,), jnp.float32),
            pltpu.SemaphoreType.DMA,
            pltpu.SemaphoreType.DMA,
        ],
        compiler_params=pltpu.CompilerParams(needs_layout_passes=False),
    )
    return run(xt, tbl).T
